# baseline (speedup 1.0000x reference)
import jax
import jax.numpy as jnp
from jax import lax
from jax.experimental import pallas as pl
from jax.experimental.pallas import tpu as pltpu
from jax.experimental.pallas import tpu_sc as plsc

_NUM_EMB = 1000000
_DIM = 64
_BATCH = 4096
_SEQ = 200

_NC = 2
_NS = 16
_NW = _NC * _NS

_TOTAL = _BATCH * _SEQ
_PER_W = _TOTAL // _NW
_CHUNK = 128
_NCHUNK = _PER_W // _CHUNK
_K = 4
_NGROUP = _NCHUNK // _K


def _body(idx_hbm, table_hbm, out_hbm, idx_v, rows_v, sem_g, sem_s):
  wid = lax.axis_index("s") * _NC + lax.axis_index("c")
  pltpu.sync_copy(idx_hbm.at[wid], idx_v)

  row0 = wid * _PER_W

  @pl.loop(0, _NGROUP)
  def _group(g):
    c0 = g * _K
    gathers = [
        pltpu.async_copy(
            table_hbm.at[idx_v.at[c0 + b]], rows_v.at[b], sem_g)
        for b in range(_K)
    ]
    for cp in gathers:
      cp.wait()
    stores = [
        pltpu.async_copy(
            rows_v.at[b],
            out_hbm.at[pl.ds(row0 + (c0 + b) * _CHUNK, _CHUNK)],
            sem_s)
        for b in range(_K)
    ]
    for cp in stores:
      cp.wait()


def kernel(x, weight):
  idx = x.astype(jnp.int32).reshape(_NW, _NCHUNK, _CHUNK)
  mesh = plsc.VectorSubcoreMesh(
      core_axis_name="c", subcore_axis_name="s",
      num_cores=_NC, num_subcores=_NS)
  out = pl.kernel(
      _body,
      out_type=jax.ShapeDtypeStruct((_TOTAL, _DIM), jnp.float32),
      mesh=mesh,
      scratch_types=[
          pltpu.VMEM((_NCHUNK, _CHUNK), jnp.int32),
          pltpu.VMEM((_K, _CHUNK, _DIM), jnp.float32),
          pltpu.SemaphoreType.DMA,
          pltpu.SemaphoreType.DMA,
      ],
      compiler_params=pltpu.CompilerParams(use_tc_tiling_on_sc=False),
  )(idx, weight)
  return out.reshape(_BATCH, _SEQ, _DIM)


# speedup vs baseline: 1.0176x; 1.0176x over previous
import jax
import jax.numpy as jnp
from jax import lax
from jax.experimental import pallas as pl
from jax.experimental.pallas import tpu as pltpu
from jax.experimental.pallas import tpu_sc as plsc

_NUM_EMB = 1000000
_DIM = 64
_BATCH = 4096
_SEQ = 200

_NC = 2
_NS = 16
_NW = _NC * _NS

_TOTAL = _BATCH * _SEQ
_PER_W = _TOTAL // _NW
_CHUNK = 128
_NCHUNK = _PER_W // _CHUNK
_K = 4
_NGROUP = _NCHUNK // _K


_NPAIR = _NGROUP // 2


def _body(idx_hbm, table_hbm, out_hbm, idx_v, rows_a, rows_b,
          sem_ga, sem_gb, sem_sa, sem_sb):
  wid = lax.axis_index("s") * _NC + lax.axis_index("c")
  pltpu.sync_copy(idx_hbm.at[wid], idx_v)

  row0 = wid * _PER_W

  def fire_gathers(g, rows, sem):
    return [
        pltpu.async_copy(table_hbm.at[idx_v.at[g * _K + b]], rows.at[b], sem)
        for b in range(_K)
    ]

  def fire_stores(g, rows, sem):
    for b in range(_K):
      pltpu.async_copy(
          rows.at[b],
          out_hbm.at[pl.ds(row0 + (g * _K + b) * _CHUNK, _CHUNK)], sem)

  def wait_stores(rows, sem):
    for b in range(_K):
      pltpu.make_async_copy(
          rows.at[b], out_hbm.at[pl.ds(row0, _CHUNK)], sem).wait()

  @pl.loop(0, _NPAIR)
  def _pair(i):
    @pl.when(i > 0)
    def _():
      wait_stores(rows_a, sem_sa)
      wait_stores(rows_b, sem_sb)
    ga = fire_gathers(2 * i, rows_a, sem_ga)
    gb = fire_gathers(2 * i + 1, rows_b, sem_gb)
    for cp in ga:
      cp.wait()
    fire_stores(2 * i, rows_a, sem_sa)
    for cp in gb:
      cp.wait()
    fire_stores(2 * i + 1, rows_b, sem_sb)

  wait_stores(rows_a, sem_sa)
  wait_stores(rows_b, sem_sb)


def kernel(x, weight):
  idx = x.astype(jnp.int32).reshape(_NW, _NCHUNK, _CHUNK)
  mesh = plsc.VectorSubcoreMesh(
      core_axis_name="c", subcore_axis_name="s",
      num_cores=_NC, num_subcores=_NS)
  out = pl.kernel(
      _body,
      out_type=jax.ShapeDtypeStruct((_TOTAL, _DIM), jnp.float32),
      mesh=mesh,
      scratch_types=[
          pltpu.VMEM((_NCHUNK, _CHUNK), jnp.int32),
          pltpu.VMEM((_K, _CHUNK, _DIM), jnp.float32),
          pltpu.VMEM((_K, _CHUNK, _DIM), jnp.float32),
          pltpu.SemaphoreType.DMA,
          pltpu.SemaphoreType.DMA,
          pltpu.SemaphoreType.DMA,
          pltpu.SemaphoreType.DMA,
      ],
      compiler_params=pltpu.CompilerParams(use_tc_tiling_on_sc=False),
  )(idx, weight)
  return out.reshape(_BATCH, _SEQ, _DIM)
